# Initial kernel scaffold; baseline (speedup 1.0000x reference)
import jax
import jax.numpy as jnp
from jax.experimental import pallas as pl


def kernel(user_id, user_age, user_text, item_id, item_cate, item_text, user_id_table, user_age_table, text_table, item_id_table, item_cate_table, u_w1, u_b1, u_w2, u_b2, i_w1, i_b1, i_w2, i_b2):
    raise NotImplementedError("write your pallas kernel here")


# speedup vs baseline: 1.8266x; 1.8266x over previous
import functools

import jax
import jax.numpy as jnp
from jax import lax
from jax.experimental import pallas as pl
from jax.experimental.pallas import tpu as pltpu
from jax.experimental.pallas import tpu_sc as plsc

B = 4096
D = 64
HIST = 50
HIST_P = 56
H1, H2 = 64, 32
NC, NS, L = 2, 16, 16
NW = NC * NS
BPW = B // NW
NCH = D // L

_mesh = plsc.VectorSubcoreMesh(
    core_axis_name="c", subcore_axis_name="s", num_cores=NC, num_subcores=NS)


@functools.partial(
    pl.kernel,
    out_type=[jax.ShapeDtypeStruct((B, D), jnp.float32)] * 6,
    mesh=_mesh,
    scratch_types=[
        pltpu.VMEM((BPW, HIST_P), jnp.int32),
        pltpu.VMEM((2, HIST_P, D), jnp.float32),
        pltpu.VMEM((BPW, D), jnp.float32),
        pltpu.VMEM((BPW,), jnp.int32),
        pltpu.VMEM((BPW, D), jnp.float32),
        pltpu.SemaphoreType.DMA,
        pltpu.SemaphoreType.DMA,
        pltpu.SemaphoreType.DMA,
    ],
    compiler_params=pltpu.CompilerParams(use_tc_tiling_on_sc=False),
)
def _sc_embed(uidx, aidx, utidx, iidx, cidx, itidx,
              uid_tab, uage_tab, text_tab, iid_tab, icate_tab,
              out_uid, out_uage, out_utx, out_iid, out_icate, out_itx,
              tidx_v, rows_v, pool_v, idbuf_v, idrows_v, sem0, sem1, semg):
    wid = lax.axis_index("s") * NC + lax.axis_index("c")
    base = wid * BPW

    def field(idx_hbm, tab, out_hbm):
        pltpu.sync_copy(idx_hbm.at[pl.ds(base, BPW)], idbuf_v)
        pltpu.async_copy(tab.at[idbuf_v], idrows_v, semg).wait()
        pltpu.sync_copy(idrows_v, out_hbm.at[pl.ds(base, BPW), :])

    field(uidx, uid_tab, out_uid)
    field(aidx, uage_tab, out_uage)
    field(iidx, iid_tab, out_iid)
    field(cidx, icate_tab, out_icate)

    def acc_row(slot, b):
        r = rows_v.at[slot]
        zero = jnp.zeros((L,), jnp.float32)

        def inner(j, accs):
            return tuple(accs[c] + r[j, pl.ds(c * L, L)] for c in range(NCH))

        accs = lax.fori_loop(0, HIST, inner, (zero,) * NCH)
        for c in range(NCH):
            pool_v[b, pl.ds(c * L, L)] = accs[c] * (1.0 / HIST)

    def tower(tidx_hbm, out_hbm):
        pltpu.sync_copy(tidx_hbm.at[pl.ds(base, BPW), :], tidx_v)
        pltpu.async_copy(text_tab.at[tidx_v.at[0]], rows_v.at[0], sem0)

        def body(k, carry):
            b0 = 2 * k
            b1 = b0 + 1
            pltpu.async_copy(text_tab.at[tidx_v.at[b1]], rows_v.at[1], sem1)
            pltpu.make_async_copy(
                text_tab.at[tidx_v.at[b0]], rows_v.at[0], sem0).wait()
            acc_row(0, b0)

            @pl.when(b1 + 1 < BPW)
            def _():
                pltpu.async_copy(
                    text_tab.at[tidx_v.at[b1 + 1]], rows_v.at[0], sem0)

            pltpu.make_async_copy(
                text_tab.at[tidx_v.at[b1]], rows_v.at[1], sem1).wait()
            acc_row(1, b1)
            return carry

        lax.fori_loop(0, BPW // 2, body, 0)
        pltpu.sync_copy(pool_v, out_hbm.at[pl.ds(base, BPW), :])

    tower(utidx, out_utx)
    tower(itidx, out_itx)


def _tc_body(euid, euage, eutx, eiid, eicate, eitx,
             uw1, ub1, uw2, ub2, iw1, ib1, iw2, ib2, out):
    f32 = jnp.float32

    def tower(e1, e2, e3, w1, b1, w2, b2):
        h = (jnp.dot(e1[...], w1[0:D], preferred_element_type=f32)
             + jnp.dot(e2[...], w1[D:2 * D], preferred_element_type=f32)
             + jnp.dot(e3[...], w1[2 * D:3 * D], preferred_element_type=f32)
             + b1[...])
        h = jnp.maximum(h, 0.0)
        o = jnp.dot(h, w2[...], preferred_element_type=f32) + b2[...]
        return jnp.maximum(o, 0.0)

    u = tower(euid, euage, eutx, uw1, ub1, uw2, ub2)
    it = tower(eiid, eicate, eitx, iw1, ib1, iw2, ib2)
    dot = jnp.sum(u * it)
    nu = jnp.sum(u * u)
    ni = jnp.sum(it * it)
    out[0, 0] = dot / (jnp.sqrt(nu) * jnp.sqrt(ni))


_tc_call = pl.pallas_call(
    _tc_body,
    out_shape=jax.ShapeDtypeStruct((1, 1), jnp.float32),
    out_specs=pl.BlockSpec(memory_space=pltpu.SMEM),
)


def kernel(user_id, user_age, user_text, item_id, item_cate, item_text,
           user_id_table, user_age_table, text_table, item_id_table,
           item_cate_table, u_w1, u_b1, u_w2, u_b2, i_w1, i_b1, i_w2, i_b2):
    uidx = user_id.reshape(B).astype(jnp.int32)
    aidx = user_age.reshape(B).astype(jnp.int32)
    iidx = item_id.reshape(B).astype(jnp.int32)
    cidx = item_cate.reshape(B).astype(jnp.int32)
    utp = jnp.pad(user_text.astype(jnp.int32), ((0, 0), (0, HIST_P - HIST)))
    itp = jnp.pad(item_text.astype(jnp.int32), ((0, 0), (0, HIST_P - HIST)))
    euid, euage, eutx, eiid, eicate, eitx = _sc_embed(
        uidx, aidx, utp, iidx, cidx, itp,
        user_id_table, user_age_table, text_table, item_id_table,
        item_cate_table)
    score = _tc_call(
        euid, euage, eutx, eiid, eicate, eitx,
        u_w1, u_b1.reshape(1, H1), u_w2, u_b2.reshape(1, H2),
        i_w1, i_b1.reshape(1, H1), i_w2, i_b2.reshape(1, H2))
    return score.reshape(())
